# baseline (speedup 1.0000x reference)
import jax
import jax.numpy as jnp
from jax.experimental import pallas as pl

Q = 8
K = 1024
D = 256
B, N = 8, 1024
M = 1024
TOKENS = B * N


def _vq_layer_kernel(r_ref, rsum_ref, esum_ref, hi_ref, stack_ref,
                     newr_ref, idx_ref, loss_ref):
    i = pl.program_id(0)

    @pl.when(i == 0)
    def _init():
        loss_ref[...] = jnp.zeros_like(loss_ref)

    r = r_ref[...]
    rsum = rsum_ref[...][:, None]
    esum = esum_ref[...]
    col = jax.lax.broadcasted_iota(jnp.int32, (M, K), 1)
    col3 = jax.lax.broadcasted_iota(jnp.int32, (M, 3 * K), 1)

    prod = jax.lax.dot_general(
        r.astype(jnp.bfloat16), hi_ref[...],
        (((1,), (1,)), ((), ())),
        preferred_element_type=jnp.float32)
    dist = (rsum - 2.0 * prod) + esum[None, :]
    mind = jnp.min(dist, axis=1, keepdims=True)
    idx = jnp.min(jnp.where(dist == mind, col, K), axis=1)
    oh3 = ((col3 & (K - 1)) == idx[:, None]).astype(jnp.bfloat16)
    quant = jax.lax.dot_general(
        oh3, stack_ref[...], (((1,), (0,)), ((), ())),
        preferred_element_type=jnp.float32)
    newr = r - quant
    loss_ref[0, :] += jnp.sum(newr * newr)
    newr_ref[...] = newr
    idx_ref[...] = idx


def _vq_layer(r, rsum, esum, hi, stack):
    return pl.pallas_call(
        _vq_layer_kernel,
        grid=(TOKENS // M,),
        in_specs=[
            pl.BlockSpec((M, D), lambda i: (i, 0)),
            pl.BlockSpec((M,), lambda i: (i,)),
            pl.BlockSpec((K,), lambda i: (0,)),
            pl.BlockSpec((K, D), lambda i: (0, 0)),
            pl.BlockSpec((3 * K, D), lambda i: (0, 0)),
        ],
        out_specs=[
            pl.BlockSpec((M, D), lambda i: (i, 0)),
            pl.BlockSpec((M,), lambda i: (i,)),
            pl.BlockSpec((1, 128), lambda i: (0, 0)),
        ],
        out_shape=[
            jax.ShapeDtypeStruct((TOKENS, D), jnp.float32),
            jax.ShapeDtypeStruct((TOKENS,), jnp.int32),
            jax.ShapeDtypeStruct((1, 128), jnp.float32),
        ],
    )(r, rsum, esum, hi, stack)


@jax.jit
def kernel(x, codebooks):
    flat = x.reshape(TOKENS, D)
    cb_hi = jax.lax.optimization_barrier(codebooks.astype(jnp.bfloat16))
    r1 = jax.lax.optimization_barrier(
        codebooks - cb_hi.astype(jnp.float32))
    cb_mid = jax.lax.optimization_barrier(r1.astype(jnp.bfloat16))
    cb_lo = (r1 - cb_mid.astype(jnp.float32)).astype(jnp.bfloat16)
    cb_stack = jnp.concatenate([cb_hi, cb_mid, cb_lo], axis=1)

    r = flat
    idx_list = []
    loss_list = []
    for q in range(Q):
        rsum = jnp.sum(r ** 2, axis=-1)
        esum = jnp.sum(codebooks[q] ** 2, axis=-1)
        r, idx, loss = _vq_layer(r, rsum, esum, cb_hi[q], cb_stack[q])
        idx_list.append(idx.reshape(B, N))
        loss_list.append(loss[0, 0])

    quantized_out = (flat - r).reshape(B, N, D)
    all_indices = jnp.stack(idx_list, axis=-1)
    all_losses = jnp.stack(loss_list) / jnp.float32(TOKENS * D)
    return quantized_out, all_indices, all_losses


# speedup vs baseline: 1.0063x; 1.0063x over previous
import jax
import jax.numpy as jnp
from jax.experimental import pallas as pl

Q = 8
K = 1024
D = 256
B, N = 8, 1024
M = 2048
TOKENS = B * N


def _vq_layer_kernel(r_ref, rsum_ref, esum_ref, hi_ref, stack_ref,
                     newr_ref, idx_ref, loss_ref):
    i = pl.program_id(0)

    @pl.when(i == 0)
    def _init():
        loss_ref[...] = jnp.zeros_like(loss_ref)

    r = r_ref[...]
    rsum = rsum_ref[...][:, None]
    esum = esum_ref[...]
    col = jax.lax.broadcasted_iota(jnp.int32, (M, K), 1)
    col3 = jax.lax.broadcasted_iota(jnp.int32, (M, 3 * K), 1)

    prod = jax.lax.dot_general(
        r.astype(jnp.bfloat16), hi_ref[...],
        (((1,), (1,)), ((), ())),
        preferred_element_type=jnp.float32)
    dist = (rsum - 2.0 * prod) + esum[None, :]
    mind = jnp.min(dist, axis=1, keepdims=True)
    idx = jnp.min(jnp.where(dist == mind, col, K), axis=1)
    oh3 = ((col3 & (K - 1)) == idx[:, None]).astype(jnp.bfloat16)
    quant = jax.lax.dot_general(
        oh3, stack_ref[...], (((1,), (0,)), ((), ())),
        preferred_element_type=jnp.float32)
    newr = r - quant
    loss_ref[0, :] += jnp.sum(newr * newr)
    newr_ref[...] = newr
    idx_ref[...] = idx


def _vq_layer(r, rsum, esum, hi, stack):
    return pl.pallas_call(
        _vq_layer_kernel,
        grid=(TOKENS // M,),
        in_specs=[
            pl.BlockSpec((M, D), lambda i: (i, 0)),
            pl.BlockSpec((M,), lambda i: (i,)),
            pl.BlockSpec((K,), lambda i: (0,)),
            pl.BlockSpec((K, D), lambda i: (0, 0)),
            pl.BlockSpec((3 * K, D), lambda i: (0, 0)),
        ],
        out_specs=[
            pl.BlockSpec((M, D), lambda i: (i, 0)),
            pl.BlockSpec((M,), lambda i: (i,)),
            pl.BlockSpec((1, 128), lambda i: (0, 0)),
        ],
        out_shape=[
            jax.ShapeDtypeStruct((TOKENS, D), jnp.float32),
            jax.ShapeDtypeStruct((TOKENS,), jnp.int32),
            jax.ShapeDtypeStruct((1, 128), jnp.float32),
        ],
    )(r, rsum, esum, hi, stack)


@jax.jit
def kernel(x, codebooks):
    flat = x.reshape(TOKENS, D)
    cb_hi = jax.lax.optimization_barrier(codebooks.astype(jnp.bfloat16))
    r1 = jax.lax.optimization_barrier(
        codebooks - cb_hi.astype(jnp.float32))
    cb_mid = jax.lax.optimization_barrier(r1.astype(jnp.bfloat16))
    cb_lo = (r1 - cb_mid.astype(jnp.float32)).astype(jnp.bfloat16)
    cb_stack = jnp.concatenate([cb_hi, cb_mid, cb_lo], axis=1)

    r = flat
    idx_list = []
    loss_list = []
    for q in range(Q):
        rsum = jnp.sum(r ** 2, axis=-1)
        esum = jnp.sum(codebooks[q] ** 2, axis=-1)
        r, idx, loss = _vq_layer(r, rsum, esum, cb_hi[q], cb_stack[q])
        idx_list.append(idx.reshape(B, N))
        loss_list.append(loss[0, 0])

    quantized_out = (flat - r).reshape(B, N, D)
    all_indices = jnp.stack(idx_list, axis=-1)
    all_losses = jnp.stack(loss_list) / jnp.float32(TOKENS * D)
    return quantized_out, all_indices, all_losses
